# baseline (speedup 1.0000x reference)
import functools

import jax
import jax.numpy as jnp
from jax import lax
from jax.experimental import pallas as pl
from jax.experimental.pallas import tpu as pltpu
from jax.experimental.pallas import tpu_sc as plsc

_NPTS = 9
_HALF = 4
_LANES = 128
_BR = 40

_SC_NC = 2
_SC_NS = 16
_SC_NW = _SC_NC * _SC_NS


def _dense_body(hw, rows_per_batch, X, Y, Z,
                scal_ref, depth_ref,
                coords_ref, dirs_ref, pts_ref, idx_ref, gidx_ref, maskf_ref):
    pid = pl.program_id(0)
    b_s = (pid * _BR) // rows_per_batch
    w = hw[1]
    row = lax.broadcasted_iota(jnp.int32, (_BR, _LANES), 0) + pid * _BR
    lane = lax.broadcasted_iota(jnp.int32, (_BR, _LANES), 1)
    p = (row % rows_per_batch) * _LANES + lane
    i_ = p // w
    j_ = p - i_ * w

    bf = lambda x: x.astype(jnp.bfloat16).astype(jnp.float32)

    zf = depth_ref[...]
    jf = j_.astype(jnp.float32)
    if_ = i_.astype(jnp.float32)
    ppx = bf(jf * zf)
    ppy = bf(if_ * zf)
    ppz = bf(zf)

    s = lambda k: scal_ref[b_s, k]
    m = [s(k) for k in range(9)]
    e = [s(9 + k) for k in range(12)]
    ox, oy, oz = s(21), s(22), s(23)
    res = s(24)

    pcx = m[0] * ppx + m[1] * ppy + m[2] * ppz
    pcy = m[3] * ppx + m[4] * ppy + m[5] * ppz
    pcz = m[6] * ppx + m[7] * ppy + m[8] * ppz

    pcxb, pcyb, pczb = bf(pcx), bf(pcy), bf(pcz)
    pwx = e[0] * pcxb + e[1] * pcyb + e[2] * pczb + e[3]
    pwy = e[4] * pcxb + e[5] * pcyb + e[6] * pczb + e[7]
    pwz = e[8] * pcxb + e[9] * pcyb + e[10] * pczb + e[11]

    coords_ref[0] = pwx
    coords_ref[1] = pwy
    coords_ref[2] = pwz

    cenx = (pwx - ox) / res
    ceny = (pwy - oy) / res
    cenz = (pwz - oz) / res
    eyex = (e[3] - ox) / res
    eyey = (e[7] - oy) / res
    eyez = (e[11] - oz) / res

    dx = cenx - eyex
    dy = ceny - eyey
    dz = cenz - eyez
    nrm = jnp.sqrt(dx * dx + dy * dy + dz * dz)
    den = jnp.maximum(nrm, jnp.float32(1e-12))
    dirx = dx / den
    diry = dy / den
    dirz = dz / den
    dirs_ref[0] = dirx
    dirs_ref[1] = diry
    dirs_ref[2] = dirz

    cen = (cenx, ceny, cenz)
    dirv = (dirx, diry, dirz)
    for k in range(_NPTS):
        off = jnp.float32(k - _HALF)
        ixs = []
        for c in range(3):
            pt = cen[c] + off * dirv[c]
            pts_ref[3 * k + c] = pt
            ix = jnp.floor(pt).astype(jnp.int32)
            idx_ref[3 * k + c] = ix
            ixs.append(ix)
        valid = ((ixs[0] >= 0) & (ixs[0] < X) &
                 (ixs[1] >= 0) & (ixs[1] < Y) &
                 (ixs[2] >= 0) & (ixs[2] < Z))
        xi = jnp.clip(ixs[0], 0, X - 1)
        yi = jnp.clip(ixs[1], 0, Y - 1)
        zi = jnp.clip(ixs[2], 0, Z - 1)
        gidx_ref[k] = (xi * (Y * Z) + yi * Z) + zi
        maskf_ref[k] = jnp.where(valid, jnp.float32(1.0), jnp.float32(0.0))


def _dense_call(scal, depth2, b, hw, X, Y, Z):
    nrows = depth2.shape[0]
    rows_per_batch = nrows // b
    grid = (nrows // _BR,)
    f32, i32 = jnp.float32, jnp.int32
    out_shape = (
        jax.ShapeDtypeStruct((3, nrows, _LANES), f32),
        jax.ShapeDtypeStruct((3, nrows, _LANES), f32),
        jax.ShapeDtypeStruct((3 * _NPTS, nrows, _LANES), f32),
        jax.ShapeDtypeStruct((3 * _NPTS, nrows, _LANES), i32),
        jax.ShapeDtypeStruct((_NPTS, nrows, _LANES), i32),
        jax.ShapeDtypeStruct((_NPTS, nrows, _LANES), f32),
    )
    plane = lambda c: pl.BlockSpec((c, _BR, _LANES), lambda g: (0, g, 0))
    return pl.pallas_call(
        functools.partial(_dense_body, hw, rows_per_batch, X, Y, Z),
        grid=grid,
        in_specs=[
            pl.BlockSpec(memory_space=pltpu.SMEM),
            pl.BlockSpec((_BR, _LANES), lambda g: (g, 0)),
        ],
        out_specs=tuple(plane(c) for c in (3, 3, 3 * _NPTS, 3 * _NPTS, _NPTS, _NPTS)),
        out_shape=out_shape,
    )(scal, depth2)


_GW = 96
_GR = 75
_GG = 15


def _sc_gather(vol_flat, gidx_flat, maskf_flat, total):
    chunk = _GR * _GW
    per_w = total // _SC_NW
    nchunks = per_w // chunk
    mesh = plsc.VectorSubcoreMesh(core_axis_name="c", subcore_axis_name="s",
                                  num_cores=_SC_NC, num_subcores=_SC_NS)

    assert nchunks % 2 == 0

    @functools.partial(
        pl.kernel,
        out_type=jax.ShapeDtypeStruct((total,), jnp.float32),
        mesh=mesh,
        scratch_types=[
            pltpu.VMEM((chunk,), jnp.int32),
            pltpu.VMEM((chunk,), jnp.int32),
            pltpu.VMEM((chunk,), jnp.float32),
            pltpu.VMEM((chunk,), jnp.float32),
            pltpu.VMEM((chunk,), jnp.float32),
            pltpu.VMEM((chunk,), jnp.float32),
            pltpu.SemaphoreType.DMA,
            pltpu.SemaphoreType.DMA,
        ],
    )
    def body(vol_hbm, gidx_hbm, maskf_hbm, out_hbm,
             idx0, idx1, vals0, vals1, mask0, mask1, sem0, sem1):
        wid = lax.axis_index("s") * _SC_NC + lax.axis_index("c")
        base = pl.multiple_of(wid * per_w, 8)

        def fire(ci, idx_v, vals_v, mask_v, sem):
            off = pl.multiple_of(base + ci * chunk, 8)
            pltpu.sync_copy(gidx_hbm.at[pl.ds(off, chunk)], idx_v)

            def group(g, c2):
                for jj in range(_GG):
                    o = pl.multiple_of((g * _GG + jj) * _GW, 8)
                    pltpu.async_copy(
                        vol_hbm.at[idx_v.at[pl.ds(o, _GW)]],
                        vals_v.at[pl.ds(o, _GW)], sem)
                return c2

            lax.fori_loop(0, _GR // _GG, group, 0)
            pltpu.sync_copy(maskf_hbm.at[pl.ds(off, chunk)], mask_v)

        def drain_and_store(ci, idx_v, vals_v, mask_v, sem):
            def group(g, c2):
                for jj in range(_GG):
                    o = pl.multiple_of((g * _GG + jj) * _GW, 8)
                    pltpu.make_async_copy(
                        vol_hbm.at[idx_v.at[pl.ds(o, _GW)]],
                        vals_v.at[pl.ds(o, _GW)], sem).wait()
                return c2

            lax.fori_loop(0, _GR // _GG, group, 0)

            def mul(i, c2):
                sl = pl.ds(i * 16, 16)
                vals_v[sl] = vals_v[sl] * mask_v[sl]
                return c2

            lax.fori_loop(0, chunk // 16, mul, 0)
            off = pl.multiple_of(base + ci * chunk, 8)
            pltpu.sync_copy(vals_v, out_hbm.at[pl.ds(off, chunk)])

        fire(0, idx0, vals0, mask0, sem0)

        def step(i, carry):
            ci = i * 2
            fire(ci + 1, idx1, vals1, mask1, sem1)
            drain_and_store(ci, idx0, vals0, mask0, sem0)

            @pl.when(ci + 2 < nchunks)
            def _():
                fire(ci + 2, idx0, vals0, mask0, sem0)

            drain_and_store(ci + 1, idx1, vals1, mask1, sem1)
            return carry

        lax.fori_loop(0, nchunks // 2, step, 0)

    return body(vol_flat, gidx_flat, maskf_flat)


def kernel(depth, extrinsics, intrinsics, feature_volume, origin, resolution):
    b, h, w = depth.shape
    n = h * w
    X, Y, Z = feature_volume.shape
    assert n % _LANES == 0
    nrows = (b * n) // _LANES
    assert (nrows // b) % _BR == 0

    def bfr(x):
        u = lax.bitcast_convert_type(x, jnp.uint32)
        r = (u + jnp.uint32(0x7FFF) + ((u >> 16) & jnp.uint32(1))) & jnp.uint32(0xFFFF0000)
        return lax.bitcast_convert_type(r, jnp.float32)

    minv = bfr(jnp.linalg.inv(intrinsics)).reshape(b, 9)
    e3 = bfr(extrinsics[:, :3, :]).reshape(b, 12)
    org = jnp.broadcast_to(origin[None, :], (b, 3))
    res = jnp.broadcast_to(
        jnp.asarray(resolution, jnp.float32)[None], (b,))[:, None]
    scal = jnp.concatenate(
        [minv, e3, org, res, jnp.zeros((b, 7), jnp.float32)], axis=1)

    depth2 = depth.reshape(nrows, _LANES)
    coords_soa, dirs_soa, pts_soa, idx_soa, gidx, maskf = _dense_call(
        scal, depth2, b, (h, w), X, Y, Z)

    total = b * n * _NPTS
    assert total % (_SC_NW * _GR * _GW) == 0
    vals = _sc_gather(feature_volume.reshape(-1), gidx.reshape(-1),
                      maskf.reshape(-1), total)

    extracted = vals.reshape(_NPTS, b, n).transpose(1, 2, 0)
    ray_pts = pts_soa.reshape(3 * _NPTS, b, n).transpose(1, 2, 0).reshape(
        b, n, _NPTS, 3)
    ray_dirs = dirs_soa.reshape(3, b, n).transpose(1, 2, 0)
    indices = idx_soa.reshape(3 * _NPTS, b, n).transpose(1, 2, 0).reshape(
        b, n, _NPTS, 3)
    coords = coords_soa.reshape(3, b, n).transpose(1, 2, 0)
    return (extracted, ray_pts, ray_dirs, depth.reshape(b, n), indices, coords)


# speedup vs baseline: 1.0167x; 1.0167x over previous
import functools

import jax
import jax.numpy as jnp
from jax import lax
from jax.experimental import pallas as pl
from jax.experimental.pallas import tpu as pltpu
from jax.experimental.pallas import tpu_sc as plsc

_NPTS = 9
_HALF = 4
_LANES = 128
_BR = 40

_SC_NC = 2
_SC_NS = 16
_SC_NW = _SC_NC * _SC_NS


def _dense_body(hw, rows_per_batch, X, Y, Z,
                scal_ref, depth_ref,
                coords_ref, dirs_ref, pts_ref, idx_ref, gidx_ref, maskf_ref):
    pid = pl.program_id(0)
    b_s = (pid * _BR) // rows_per_batch
    w = hw[1]
    row = lax.broadcasted_iota(jnp.int32, (_BR, _LANES), 0) + pid * _BR
    lane = lax.broadcasted_iota(jnp.int32, (_BR, _LANES), 1)
    p = (row % rows_per_batch) * _LANES + lane
    i_ = p // w
    j_ = p - i_ * w

    bf = lambda x: x.astype(jnp.bfloat16).astype(jnp.float32)

    zf = depth_ref[...]
    jf = j_.astype(jnp.float32)
    if_ = i_.astype(jnp.float32)
    ppx = bf(jf * zf)
    ppy = bf(if_ * zf)
    ppz = bf(zf)

    s = lambda k: scal_ref[b_s, k]
    m = [s(k) for k in range(9)]
    e = [s(9 + k) for k in range(12)]
    ox, oy, oz = s(21), s(22), s(23)
    res = s(24)

    pcx = m[0] * ppx + m[1] * ppy + m[2] * ppz
    pcy = m[3] * ppx + m[4] * ppy + m[5] * ppz
    pcz = m[6] * ppx + m[7] * ppy + m[8] * ppz

    pcxb, pcyb, pczb = bf(pcx), bf(pcy), bf(pcz)
    pwx = e[0] * pcxb + e[1] * pcyb + e[2] * pczb + e[3]
    pwy = e[4] * pcxb + e[5] * pcyb + e[6] * pczb + e[7]
    pwz = e[8] * pcxb + e[9] * pcyb + e[10] * pczb + e[11]

    coords_ref[0] = pwx
    coords_ref[1] = pwy
    coords_ref[2] = pwz

    cenx = (pwx - ox) / res
    ceny = (pwy - oy) / res
    cenz = (pwz - oz) / res
    eyex = (e[3] - ox) / res
    eyey = (e[7] - oy) / res
    eyez = (e[11] - oz) / res

    dx = cenx - eyex
    dy = ceny - eyey
    dz = cenz - eyez
    nrm = jnp.sqrt(dx * dx + dy * dy + dz * dz)
    den = jnp.maximum(nrm, jnp.float32(1e-12))
    dirx = dx / den
    diry = dy / den
    dirz = dz / den
    dirs_ref[0] = dirx
    dirs_ref[1] = diry
    dirs_ref[2] = dirz

    cen = (cenx, ceny, cenz)
    dirv = (dirx, diry, dirz)
    for k in range(_NPTS):
        off = jnp.float32(k - _HALF)
        ixs = []
        for c in range(3):
            pt = cen[c] + off * dirv[c]
            pts_ref[3 * k + c] = pt
            ix = jnp.floor(pt).astype(jnp.int32)
            idx_ref[3 * k + c] = ix
            ixs.append(ix)
        valid = ((ixs[0] >= 0) & (ixs[0] < X) &
                 (ixs[1] >= 0) & (ixs[1] < Y) &
                 (ixs[2] >= 0) & (ixs[2] < Z))
        xi = jnp.clip(ixs[0], 0, X - 1)
        yi = jnp.clip(ixs[1], 0, Y - 1)
        zi = jnp.clip(ixs[2], 0, Z - 1)
        gidx_ref[k] = (xi * (Y * Z) + yi * Z) + zi
        maskf_ref[k] = jnp.where(valid, jnp.float32(1.0), jnp.float32(0.0))


def _dense_call(scal, depth2, b, hw, X, Y, Z):
    nrows = depth2.shape[0]
    rows_per_batch = nrows // b
    grid = (nrows // _BR,)
    f32, i32 = jnp.float32, jnp.int32
    out_shape = (
        jax.ShapeDtypeStruct((3, nrows, _LANES), f32),
        jax.ShapeDtypeStruct((3, nrows, _LANES), f32),
        jax.ShapeDtypeStruct((3 * _NPTS, nrows, _LANES), f32),
        jax.ShapeDtypeStruct((3 * _NPTS, nrows, _LANES), i32),
        jax.ShapeDtypeStruct((_NPTS, nrows, _LANES), i32),
        jax.ShapeDtypeStruct((_NPTS, nrows, _LANES), f32),
    )
    plane = lambda c: pl.BlockSpec((c, _BR, _LANES), lambda g: (0, g, 0))
    return pl.pallas_call(
        functools.partial(_dense_body, hw, rows_per_batch, X, Y, Z),
        grid=grid,
        in_specs=[
            pl.BlockSpec(memory_space=pltpu.SMEM),
            pl.BlockSpec((_BR, _LANES), lambda g: (g, 0)),
        ],
        out_specs=tuple(plane(c) for c in (3, 3, 3 * _NPTS, 3 * _NPTS, _NPTS, _NPTS)),
        out_shape=out_shape,
    )(scal, depth2)


_GW = 96
_GR = 45
_GG = 15


def _sc_gather(vol_flat, gidx_flat, maskf_flat, total):
    chunk = _GR * _GW
    per_w = total // _SC_NW
    nchunks = per_w // chunk
    mesh = plsc.VectorSubcoreMesh(core_axis_name="c", subcore_axis_name="s",
                                  num_cores=_SC_NC, num_subcores=_SC_NS)

    assert nchunks % 2 == 0

    @functools.partial(
        pl.kernel,
        out_type=jax.ShapeDtypeStruct((total,), jnp.float32),
        mesh=mesh,
        scratch_types=[
            pltpu.VMEM((chunk,), jnp.int32),
            pltpu.VMEM((chunk,), jnp.int32),
            pltpu.VMEM((chunk,), jnp.float32),
            pltpu.VMEM((chunk,), jnp.float32),
            pltpu.VMEM((chunk,), jnp.float32),
            pltpu.VMEM((chunk,), jnp.float32),
            pltpu.SemaphoreType.DMA,
            pltpu.SemaphoreType.DMA,
        ],
    )
    def body(vol_hbm, gidx_hbm, maskf_hbm, out_hbm,
             idx0, idx1, vals0, vals1, mask0, mask1, sem0, sem1):
        wid = lax.axis_index("s") * _SC_NC + lax.axis_index("c")
        base = pl.multiple_of(wid * per_w, 8)

        def fire(ci, idx_v, vals_v, mask_v, sem):
            off = pl.multiple_of(base + ci * chunk, 8)
            pltpu.sync_copy(gidx_hbm.at[pl.ds(off, chunk)], idx_v)

            def group(g, c2):
                for jj in range(_GG):
                    o = pl.multiple_of((g * _GG + jj) * _GW, 8)
                    pltpu.async_copy(
                        vol_hbm.at[idx_v.at[pl.ds(o, _GW)]],
                        vals_v.at[pl.ds(o, _GW)], sem)
                return c2

            lax.fori_loop(0, _GR // _GG, group, 0)
            pltpu.sync_copy(maskf_hbm.at[pl.ds(off, chunk)], mask_v)

        def drain_and_store(ci, idx_v, vals_v, mask_v, sem):
            def group(g, c2):
                for jj in range(_GG):
                    o = pl.multiple_of((g * _GG + jj) * _GW, 8)
                    pltpu.make_async_copy(
                        vol_hbm.at[idx_v.at[pl.ds(o, _GW)]],
                        vals_v.at[pl.ds(o, _GW)], sem).wait()
                return c2

            lax.fori_loop(0, _GR // _GG, group, 0)

            def mul(i, c2):
                sl = pl.ds(i * 16, 16)
                vals_v[sl] = vals_v[sl] * mask_v[sl]
                return c2

            lax.fori_loop(0, chunk // 16, mul, 0)
            off = pl.multiple_of(base + ci * chunk, 8)
            pltpu.sync_copy(vals_v, out_hbm.at[pl.ds(off, chunk)])

        fire(0, idx0, vals0, mask0, sem0)

        def step(i, carry):
            ci = i * 2
            fire(ci + 1, idx1, vals1, mask1, sem1)
            drain_and_store(ci, idx0, vals0, mask0, sem0)

            @pl.when(ci + 2 < nchunks)
            def _():
                fire(ci + 2, idx0, vals0, mask0, sem0)

            drain_and_store(ci + 1, idx1, vals1, mask1, sem1)
            return carry

        lax.fori_loop(0, nchunks // 2, step, 0)

    return body(vol_flat, gidx_flat, maskf_flat)


def kernel(depth, extrinsics, intrinsics, feature_volume, origin, resolution):
    b, h, w = depth.shape
    n = h * w
    X, Y, Z = feature_volume.shape
    assert n % _LANES == 0
    nrows = (b * n) // _LANES
    assert (nrows // b) % _BR == 0

    def bfr(x):
        u = lax.bitcast_convert_type(x, jnp.uint32)
        r = (u + jnp.uint32(0x7FFF) + ((u >> 16) & jnp.uint32(1))) & jnp.uint32(0xFFFF0000)
        return lax.bitcast_convert_type(r, jnp.float32)

    minv = bfr(jnp.linalg.inv(intrinsics)).reshape(b, 9)
    e3 = bfr(extrinsics[:, :3, :]).reshape(b, 12)
    org = jnp.broadcast_to(origin[None, :], (b, 3))
    res = jnp.broadcast_to(
        jnp.asarray(resolution, jnp.float32)[None], (b,))[:, None]
    scal = jnp.concatenate(
        [minv, e3, org, res, jnp.zeros((b, 7), jnp.float32)], axis=1)

    depth2 = depth.reshape(nrows, _LANES)
    coords_soa, dirs_soa, pts_soa, idx_soa, gidx, maskf = _dense_call(
        scal, depth2, b, (h, w), X, Y, Z)

    total = b * n * _NPTS
    assert total % (_SC_NW * _GR * _GW) == 0
    vals = _sc_gather(feature_volume.reshape(-1), gidx.reshape(-1),
                      maskf.reshape(-1), total)

    extracted = vals.reshape(_NPTS, b, n).transpose(1, 2, 0)
    ray_pts = pts_soa.reshape(3 * _NPTS, b, n).transpose(1, 2, 0).reshape(
        b, n, _NPTS, 3)
    ray_dirs = dirs_soa.reshape(3, b, n).transpose(1, 2, 0)
    indices = idx_soa.reshape(3 * _NPTS, b, n).transpose(1, 2, 0).reshape(
        b, n, _NPTS, 3)
    coords = coords_soa.reshape(3, b, n).transpose(1, 2, 0)
    return (extracted, ray_pts, ray_dirs, depth.reshape(b, n), indices, coords)
